# baseline (speedup 1.0000x reference)
import jax
import jax.numpy as jnp
from jax.experimental import pallas as pl

_N_CLASSES = 1000
_ROWS = 4096 * 26
_BLOCK_R = 256


def _onehot_block(idx_ref, out_ref):
    idx = idx_ref[...]
    iota = jax.lax.broadcasted_iota(jnp.int32, (_BLOCK_R, _N_CLASSES), 1)
    out_ref[...] = (iota == idx).astype(jnp.float32)


def kernel(input):
    flat = input.reshape(_ROWS, 1).astype(jnp.int32)
    out = pl.pallas_call(
        _onehot_block,
        grid=(_ROWS // _BLOCK_R,),
        in_specs=[pl.BlockSpec((_BLOCK_R, 1), lambda i: (i, 0))],
        out_specs=pl.BlockSpec((_BLOCK_R, _N_CLASSES), lambda i: (i, 0)),
        out_shape=jax.ShapeDtypeStruct((_ROWS, _N_CLASSES), jnp.float32),
    )(flat)
    return out.reshape(4096, 26, _N_CLASSES)


# speedup vs baseline: 1.7251x; 1.7251x over previous
import jax
import jax.numpy as jnp
from jax.experimental import pallas as pl

_N_CLASSES = 1000
_B, _T = 4096, 26
_BLOCK = 64


def _onehot_block(idx_ref, out_ref):
    idx = idx_ref[...]
    iota = jax.lax.broadcasted_iota(jnp.int32, (_BLOCK, _T, _N_CLASSES), 2)
    out_ref[...] = (iota == idx[:, :, None]).astype(jnp.float32)


def kernel(input):
    return pl.pallas_call(
        _onehot_block,
        grid=(_B // _BLOCK,),
        in_specs=[pl.BlockSpec((_BLOCK, _T), lambda i: (i, 0))],
        out_specs=pl.BlockSpec((_BLOCK, _T, _N_CLASSES), lambda i: (i, 0, 0)),
        out_shape=jax.ShapeDtypeStruct((_B, _T, _N_CLASSES), jnp.float32),
    )(input.astype(jnp.int32))
